# baseline (speedup 1.0000x reference)
import functools

import jax
import jax.numpy as jnp
from jax import lax
from jax.experimental import pallas as pl
from jax.experimental.pallas import tpu as pltpu
from jax.experimental.pallas import tpu_sc as plsc


def kernel(x, embd):
    T = x.shape[1]
    R, D = embd.shape
    info = plsc.get_sparse_core_info()
    nw = info.num_cores * info.num_subcores
    rows_per = T // nw

    mesh = plsc.VectorSubcoreMesh(core_axis_name="c", subcore_axis_name="s")

    @functools.partial(
        pl.kernel,
        mesh=mesh,
        out_type=jax.ShapeDtypeStruct((T, D), embd.dtype),
    )
    def copy_rows(embd_hbm, out_hbm):
        wid = lax.axis_index("s") * info.num_cores + lax.axis_index("c")
        base = wid * rows_per
        pltpu.sync_copy(
            embd_hbm.at[pl.ds(base, rows_per)],
            out_hbm.at[pl.ds(base, rows_per)],
        )

    return copy_rows(embd)


# speedup vs baseline: 1.0017x; 1.0017x over previous
import functools

import jax
import jax.numpy as jnp
from jax import lax
from jax.experimental import pallas as pl
from jax.experimental.pallas import tpu as pltpu
from jax.experimental.pallas import tpu_sc as plsc


def kernel(x, embd):
    T = x.shape[1]
    R, D = embd.shape
    info = plsc.get_sparse_core_info()
    nw = info.num_cores * info.num_subcores
    rows_per = T // nw

    mesh = plsc.VectorSubcoreMesh(core_axis_name="c", subcore_axis_name="s")

    n_chunks = 8
    chunk = rows_per // n_chunks

    @functools.partial(
        pl.kernel,
        mesh=mesh,
        out_type=jax.ShapeDtypeStruct((T, D), embd.dtype),
        scratch_types=[pltpu.SemaphoreType.DMA],
    )
    def copy_rows(embd_hbm, out_hbm, sem):
        wid = lax.axis_index("s") * info.num_cores + lax.axis_index("c")
        base = wid * rows_per
        copies = []
        for j in range(n_chunks):
            copies.append(
                pltpu.async_copy(
                    embd_hbm.at[pl.ds(base + j * chunk, chunk)],
                    out_hbm.at[pl.ds(base + j * chunk, chunk)],
                    sem,
                )
            )
        for c in copies:
            c.wait()

    return copy_rows(embd)


# speedup vs baseline: 24.3089x; 24.2665x over previous
import functools

import jax
import jax.numpy as jnp
from jax import lax
from jax.experimental import pallas as pl
from jax.experimental.pallas import tpu as pltpu
from jax.experimental.pallas import tpu_sc as plsc


def kernel(x, embd):
    T = x.shape[1]
    R, D = embd.shape
    info = plsc.get_sparse_core_info()
    nw = info.num_cores * info.num_subcores
    rows_per = T // nw

    n_buf = 2
    chunk = 32
    n_chunks = rows_per // chunk

    mesh = plsc.VectorSubcoreMesh(core_axis_name="c", subcore_axis_name="s")

    @functools.partial(
        pl.kernel,
        mesh=mesh,
        out_type=jax.ShapeDtypeStruct((T, D), embd.dtype),
        scratch_types=[
            pltpu.VMEM((n_buf, chunk, D), embd.dtype),
            pltpu.SemaphoreType.DMA((n_buf,)),
            pltpu.SemaphoreType.DMA((n_buf,)),
        ],
    )
    def copy_rows(embd_hbm, out_hbm, buf, in_sem, out_sem):
        wid = lax.axis_index("s") * info.num_cores + lax.axis_index("c")
        base = wid * rows_per

        def gather(g):
            return pltpu.async_copy(
                embd_hbm.at[pl.ds(base + g * chunk, chunk)],
                buf.at[g % n_buf],
                in_sem.at[g % n_buf],
            )

        def scatter(g):
            return pltpu.async_copy(
                buf.at[g % n_buf],
                out_hbm.at[pl.ds(base + g * chunk, chunk)],
                out_sem.at[g % n_buf],
            )

        gathers = [None] * n_chunks
        scatters = [None] * n_chunks
        for g in range(n_buf):
            gathers[g] = gather(g)
        for g in range(n_chunks):
            gathers[g].wait()
            scatters[g] = scatter(g)
            if g + n_buf < n_chunks:
                scatters[g].wait()
                gathers[g + n_buf] = gather(g + n_buf)
        for g in range(n_chunks - n_buf, n_chunks):
            scatters[g].wait()

    return copy_rows(embd)


# speedup vs baseline: 25.1265x; 1.0336x over previous
import functools

import jax
import jax.numpy as jnp
from jax import lax
from jax.experimental import pallas as pl
from jax.experimental.pallas import tpu as pltpu
from jax.experimental.pallas import tpu_sc as plsc


def kernel(x, embd):
    T = x.shape[1]
    R, D = embd.shape
    info = plsc.get_sparse_core_info()
    nw = info.num_cores * info.num_subcores
    rows_per = T // nw

    n_buf = 3
    chunk = 32
    n_chunks = rows_per // chunk

    mesh = plsc.VectorSubcoreMesh(core_axis_name="c", subcore_axis_name="s")

    @functools.partial(
        pl.kernel,
        mesh=mesh,
        out_type=jax.ShapeDtypeStruct((T, D), embd.dtype),
        scratch_types=[
            pltpu.VMEM((n_buf, chunk, D), embd.dtype),
            pltpu.SemaphoreType.DMA((n_buf,)),
            pltpu.SemaphoreType.DMA((n_buf,)),
        ],
    )
    def copy_rows(embd_hbm, out_hbm, buf, in_sem, out_sem):
        wid = lax.axis_index("s") * info.num_cores + lax.axis_index("c")
        base = wid * rows_per

        def gather(g):
            return pltpu.async_copy(
                embd_hbm.at[pl.ds(base + g * chunk, chunk)],
                buf.at[g % n_buf],
                in_sem.at[g % n_buf],
            )

        def scatter(g):
            return pltpu.async_copy(
                buf.at[g % n_buf],
                out_hbm.at[pl.ds(base + g * chunk, chunk)],
                out_sem.at[g % n_buf],
            )

        gathers = [None] * n_chunks
        scatters = [None] * n_chunks
        for g in range(n_buf):
            gathers[g] = gather(g)
        for g in range(n_chunks):
            gathers[g].wait()
            scatters[g] = scatter(g)
            if g + n_buf < n_chunks:
                scatters[g].wait()
                gathers[g + n_buf] = gather(g + n_buf)
        for g in range(n_chunks - n_buf, n_chunks):
            scatters[g].wait()

    return copy_rows(embd)
